# speedup vs baseline: 2.4210x; 2.4210x over previous
import jax
import jax.numpy as jnp
from jax import lax
from jax.experimental import pallas as pl
from jax.experimental.pallas import tpu as pltpu
from jax.experimental.pallas import tpu_sc as plsc

B = 4
H = 512
W = 512
NM = 2048
NU = 1024
NC = 2
NS = 16
TILES_PER_SAMPLE = (NC * NS) // B
M_PER_TILE = NM // TILES_PER_SAMPLE
U_PER_TILE = NU // TILES_PER_SAMPLE
PER_TILE = 2 * M_PER_TILE + 2 * U_PER_TILE
CHUNK = 128


def _tile_body(pred_hbm, tgt_hbm,
               pmb, pmd, tmb, tmd, pub, pud, tub, tud,
               out_hbm,
               ysp, xsp, yst, xst, pidx, tidx, vp, vt, part, sem):
    cid = lax.axis_index("c")
    sid = lax.axis_index("s")
    wid = cid * NS + sid
    b = wid // TILES_PER_SAMPLE
    q = wid % TILES_PER_SAMPLE
    mo = q * M_PER_TILE
    uo = q * U_PER_TILE

    copies = []
    for (ybuf, xbuf, cb, cd, ub, ud) in (
        (ysp, xsp, pmb, pmd, pub, pud),
        (yst, xst, tmb, tmd, tub, tud),
    ):
        for (buf, dim) in ((ybuf, 0), (xbuf, 1)):
            copies.append(pltpu.async_copy(
                cb.at[b, dim, pl.ds(mo, M_PER_TILE)],
                buf.at[pl.ds(0, M_PER_TILE)], sem))
            copies.append(pltpu.async_copy(
                cd.at[b, dim, pl.ds(mo, M_PER_TILE)],
                buf.at[pl.ds(M_PER_TILE, M_PER_TILE)], sem))
            copies.append(pltpu.async_copy(
                ub.at[b, dim, pl.ds(uo, U_PER_TILE)],
                buf.at[pl.ds(2 * M_PER_TILE, U_PER_TILE)], sem))
            copies.append(pltpu.async_copy(
                ud.at[b, dim, pl.ds(uo, U_PER_TILE)],
                buf.at[pl.ds(2 * M_PER_TILE + U_PER_TILE, U_PER_TILE)], sem))
    for c in copies:
        c.wait()

    boff = b * (H * W)
    for i in range(PER_TILE // 16):
        o = i * 16
        pidx[pl.ds(o, 16)] = boff + ysp[pl.ds(o, 16)] * W + xsp[pl.ds(o, 16)]
        tidx[pl.ds(o, 16)] = boff + yst[pl.ds(o, 16)] * W + xst[pl.ds(o, 16)]

    gathers = []
    for c in range(PER_TILE // CHUNK):
        o = c * CHUNK
        gathers.append(pltpu.async_copy(
            pred_hbm.at[pidx.at[pl.ds(o, CHUNK)]], vp.at[pl.ds(o, CHUNK)], sem))
        gathers.append(pltpu.async_copy(
            tgt_hbm.at[tidx.at[pl.ds(o, CHUNK)]], vt.at[pl.ds(o, CHUNK)], sem))
    for g in gathers:
        g.wait()

    acc_m = jnp.zeros((16,), jnp.float32)
    for i in range(2 * M_PER_TILE // 16):
        o = i * 16
        d = vp[pl.ds(o, 16)] - vt[pl.ds(o, 16)]
        acc_m = acc_m + d * d
    acc_u = jnp.zeros((16,), jnp.float32)
    for i in range(U_PER_TILE // 16):
        o = 2 * M_PER_TILE + i * 16
        d = vp[pl.ds(o, 16)] - vp[pl.ds(o + U_PER_TILE, 16)]
        e = vt[pl.ds(o, 16)] - vt[pl.ds(o + U_PER_TILE, 16)]
        acc_u = acc_u + d * d + e * e
    part[...] = (acc_m * 2.0 + acc_u) * (1.0 / B)
    pltpu.sync_copy(part, out_hbm.at[wid])


@jax.jit
def kernel(input, target, pred_matched_birth, pred_matched_death,
           tgt_matched_birth, tgt_matched_death,
           pred_unmatched_birth, pred_unmatched_death,
           tgt_unmatched_birth, tgt_unmatched_death):
    pred_flat = input.reshape(B * H * W)
    tgt_flat = target.reshape(B * H * W)
    coords = [c.transpose(0, 2, 1) for c in (
        pred_matched_birth, pred_matched_death,
        tgt_matched_birth, tgt_matched_death,
        pred_unmatched_birth, pred_unmatched_death,
        tgt_unmatched_birth, tgt_unmatched_death)]

    mesh = plsc.VectorSubcoreMesh(core_axis_name="c", subcore_axis_name="s")
    run = pl.kernel(
        _tile_body,
        out_type=jax.ShapeDtypeStruct((NC * NS, 16), jnp.float32),
        mesh=mesh,
        scratch_types=[
            pltpu.VMEM((PER_TILE,), jnp.int32),
            pltpu.VMEM((PER_TILE,), jnp.int32),
            pltpu.VMEM((PER_TILE,), jnp.int32),
            pltpu.VMEM((PER_TILE,), jnp.int32),
            pltpu.VMEM((PER_TILE,), jnp.int32),
            pltpu.VMEM((PER_TILE,), jnp.int32),
            pltpu.VMEM((PER_TILE,), jnp.float32),
            pltpu.VMEM((PER_TILE,), jnp.float32),
            pltpu.VMEM((16,), jnp.float32),
            pltpu.SemaphoreType.DMA,
        ],
    )
    parts = run(pred_flat, tgt_flat, *coords)
    return jnp.sum(parts)
